# baseline (speedup 1.0000x reference)
import functools
import jax
import jax.numpy as jnp
from jax import lax
from jax.experimental import pallas as pl
from jax.experimental.pallas import tpu as pltpu
from jax.experimental.pallas import tpu_sc as plsc


def _make_sc_kernel(NR, S, D, R):
    info = plsc.get_sparse_core_info()
    NC, NS = info.num_cores, info.num_subcores
    NW = NC * NS
    rows_w = NR // NW
    n_chunks = rows_w // R
    mesh = plsc.VectorSubcoreMesh(core_axis_name="c", subcore_axis_name="s")

    @functools.partial(
        pl.kernel,
        mesh=mesh,
        out_type=jax.ShapeDtypeStruct((NR, D), jnp.float32),
        scratch_types=[
            pltpu.VMEM((R, D), jnp.float32),
            pltpu.VMEM((R, D), jnp.float32),
        ],
    )
    def k(x_hbm, pos_hbm, out_hbm, bufx, bufp):
        wid = lax.axis_index("s") * NC + lax.axis_index("c")
        base = wid * rows_w
        pos_base = base % S
        for c in range(n_chunks):
            r0 = base + c * R
            p0 = pos_base + c * R
            pltpu.sync_copy(x_hbm.at[pl.ds(r0, R)], bufx)
            pltpu.sync_copy(pos_hbm.at[pl.ds(p0, R)], bufp)

            def body(r, carry):
                for j in range(D // 16):
                    sl = pl.ds(j * 16, 16)
                    plsc.addupdate(bufx.at[r, sl], bufp[r, sl])
                return carry

            lax.fori_loop(0, R, body, 0)
            pltpu.sync_copy(bufx, out_hbm.at[pl.ds(r0, R)])

    return k


def kernel(x, pos_table):
    B, S, D = x.shape
    NR = B * S
    x2 = x.reshape(NR, D)
    out = _make_sc_kernel(NR, S, D, 32)(x2, pos_table)
    return out.reshape(B, S, D)


# speedup vs baseline: 1.4074x; 1.4074x over previous
import functools
import jax
import jax.numpy as jnp
from jax import lax
from jax.experimental import pallas as pl
from jax.experimental.pallas import tpu as pltpu
from jax.experimental.pallas import tpu_sc as plsc


def _make_sc_kernel(B, S, D, R):
    info = plsc.get_sparse_core_info()
    NC, NS = info.num_cores, info.num_subcores
    NW = NC * NS
    P = S // NW
    per_b = P // R
    n_chunks = B * per_b
    assert n_chunks % 2 == 0
    mesh = plsc.VectorSubcoreMesh(core_axis_name="c", subcore_axis_name="s")

    @functools.partial(
        pl.kernel,
        mesh=mesh,
        out_type=jax.ShapeDtypeStruct((B * S, D), jnp.float32),
        scratch_types=[
            pltpu.VMEM((P, D), jnp.float32),
            pltpu.VMEM((R, D), jnp.float32),
            pltpu.VMEM((R, D), jnp.float32),
            pltpu.SemaphoreType.DMA,
            pltpu.SemaphoreType.DMA,
            pltpu.SemaphoreType.DMA,
            pltpu.SemaphoreType.DMA,
        ],
    )
    def k(x_hbm, pos_hbm, out_hbm, posbuf, b0, b1, si0, si1, so0, so1):
        wid = lax.axis_index("s") * NC + lax.axis_index("c")
        pos0 = wid * P
        pltpu.sync_copy(pos_hbm.at[pl.ds(pos0, P)], posbuf)

        def row0(i):
            return (i // per_b) * S + pos0 + (i % per_b) * R

        def start_in(i, buf, sem):
            pltpu.async_copy(x_hbm.at[pl.ds(row0(i), R)], buf, sem)

        def wait_in(buf, sem):
            pltpu.make_async_copy(x_hbm.at[pl.ds(pos0, R)], buf, sem).wait()

        def start_out(i, buf, sem):
            pltpu.async_copy(buf, out_hbm.at[pl.ds(row0(i), R)], sem)

        def wait_out(buf, sem):
            pltpu.make_async_copy(x_hbm.at[pl.ds(pos0, R)], buf, sem).wait()

        def accum(buf, i):
            c = i % per_b

            def body(r, carry):
                for j in range(D // 16):
                    sl = pl.ds(j * 16, 16)
                    plsc.addupdate(buf.at[r, sl], posbuf[c * R + r, sl])
                return carry

            lax.fori_loop(0, R, body, 0)

        start_in(0, b0, si0)

        def outer(t, carry):
            i = 2 * t
            wait_in(b0, si0)

            @pl.when(t > 0)
            def _():
                wait_out(b1, so1)

            start_in(i + 1, b1, si1)
            accum(b0, i)
            start_out(i, b0, so0)

            wait_in(b1, si1)

            @pl.when(t + 1 < n_chunks // 2)
            def _():
                wait_out(b0, so0)
                start_in(i + 2, b0, si0)

            accum(b1, i + 1)
            start_out(i + 1, b1, so1)
            return carry

        lax.fori_loop(0, n_chunks // 2, outer, 0)
        wait_out(b0, so0)
        wait_out(b1, so1)

    return k


def kernel(x, pos_table):
    B, S, D = x.shape
    x2 = x.reshape(B * S, D)
    out = _make_sc_kernel(B, S, D, 16)(x2, pos_table)
    return out.reshape(B, S, D)


# speedup vs baseline: 5.5209x; 3.9228x over previous
import jax
import jax.numpy as jnp
from jax.experimental import pallas as pl


def _add_kernel(x_ref, pos_ref, o_ref):
    o_ref[...] = x_ref[...] + pos_ref[...]


def kernel(x, pos_table):
    B, S, D = x.shape
    BS = 2048
    x2 = x.reshape(B * S, D)
    nS = S // BS
    out = pl.pallas_call(
        _add_kernel,
        grid=(nS, B),
        in_specs=[
            pl.BlockSpec((BS, D), lambda s, b: (b * nS + s, 0)),
            pl.BlockSpec((BS, D), lambda s, b: (s, 0)),
        ],
        out_specs=pl.BlockSpec((BS, D), lambda s, b: (b * nS + s, 0)),
        out_shape=jax.ShapeDtypeStruct((B * S, D), x.dtype),
    )(x2, pos_table)
    return out.reshape(B, S, D)


# speedup vs baseline: 5.5317x; 1.0020x over previous
import jax
import jax.numpy as jnp
from jax.experimental import pallas as pl


def _add_kernel(x_ref, pos_ref, o_ref):
    o_ref[...] = x_ref[...] + pos_ref[...]


def kernel(x, pos_table):
    B, S, D = x.shape
    x2 = x.reshape(B * S, D)
    out = pl.pallas_call(
        _add_kernel,
        grid=(B,),
        in_specs=[
            pl.BlockSpec((S, D), lambda b: (b, 0)),
            pl.BlockSpec((S, D), lambda b: (0, 0)),
        ],
        out_specs=pl.BlockSpec((S, D), lambda b: (b, 0)),
        out_shape=jax.ShapeDtypeStruct((B * S, D), x.dtype),
    )(x2, pos_table)
    return out.reshape(B, S, D)
